# baseline (speedup 1.0000x reference)
import functools

import jax
import jax.numpy as jnp
from jax import lax
from jax.experimental import pallas as pl
from jax.experimental.pallas import tpu as pltpu
from jax.experimental.pallas import tpu_sc as plsc

N_EMB = 1024
EMB_DIM = 64
BETA = 0.25
N_ROWS = 8192
BLK = 512
GRID = N_ROWS // BLK

SC_NC = 2
SC_NS = 16
SC_NW = SC_NC * SC_NS
ROWS_PER_W = N_ROWS // SC_NW


def _sc_gather_body(idx_hbm, emb_hbm, out_hbm, idx_v, rows_v, sem):
    wid = lax.axis_index("s") * SC_NC + lax.axis_index("c")
    base = wid * ROWS_PER_W
    pltpu.sync_copy(idx_hbm.at[pl.ds(base, ROWS_PER_W)], idx_v)
    pltpu.async_copy(emb_hbm.at[idx_v], rows_v, sem).wait()
    pltpu.sync_copy(rows_v, out_hbm.at[pl.ds(base, ROWS_PER_W)])


_sc_gather = functools.partial(
    pl.kernel,
    mesh=plsc.VectorSubcoreMesh(core_axis_name="c", subcore_axis_name="s"),
    out_type=jax.ShapeDtypeStruct((N_ROWS, 128), jnp.float32),
    scratch_types=[
        pltpu.VMEM((ROWS_PER_W,), jnp.int32),
        pltpu.VMEM((ROWS_PER_W, 128), jnp.float32),
        pltpu.SemaphoreType.DMA,
    ],
)(_sc_gather_body)


def _tc_body(z_ref, embt_ref,
             dist_ref, idx_ref, counts_ref, loss_ref, perp_ref,
             minsum_ref):
    i = pl.program_id(0)
    z = z_ref[...]
    et = embt_ref[...]
    z2 = jnp.sum(z * z, axis=1, keepdims=True)
    e2 = jnp.sum(et * et, axis=0, keepdims=True)
    d = (z2 + e2) - 2.0 * jnp.dot(z, et, preferred_element_type=jnp.float32)
    dist_ref[...] = d

    mind = jnp.min(d, axis=1)
    iota = jax.lax.broadcasted_iota(jnp.int32, (BLK, N_EMB), 1)
    idx = jnp.min(jnp.where(d == mind[:, None], iota, N_EMB),
                  axis=1).astype(jnp.int32)
    idx_ref[...] = idx.reshape(1, 1, BLK)

    onehot = (iota == idx[:, None]).astype(jnp.float32)
    cnt = jnp.sum(onehot, axis=0, keepdims=True)

    @pl.when(i == 0)
    def _init():
        counts_ref[...] = jnp.zeros((1, N_EMB), jnp.float32)
        minsum_ref[0, 0] = 0.0

    counts_ref[...] += cnt
    minsum_ref[0, 0] += jnp.sum(mind)

    @pl.when(i == GRID - 1)
    def _final():
        loss = BETA * minsum_ref[0, 0] / float(N_ROWS * EMB_DIM)
        loss_ref[...] = jnp.full((1, 1), loss, jnp.float32)
        p = counts_ref[...] / float(N_ROWS)
        perp = jnp.exp(-jnp.sum(p * jnp.log(p + 1e-10)))
        perp_ref[...] = jnp.full((1, 1), perp, jnp.float32)


def kernel(z_e, embedding):
    B, D, H, W = z_e.shape
    z_flat = jnp.transpose(z_e, (0, 2, 3, 1)).reshape(N_ROWS, EMB_DIM)
    emb_t = embedding.T

    out_shapes = (
        jax.ShapeDtypeStruct((N_ROWS, N_EMB), jnp.float32),
        jax.ShapeDtypeStruct((GRID, 1, BLK), jnp.int32),
        jax.ShapeDtypeStruct((1, N_EMB), jnp.float32),
        jax.ShapeDtypeStruct((1, 1), jnp.float32),
        jax.ShapeDtypeStruct((1, 1), jnp.float32),
    )
    dist, idx3, counts, loss, perp = pl.pallas_call(
        _tc_body,
        grid=(GRID,),
        in_specs=[
            pl.BlockSpec((BLK, EMB_DIM), lambda i: (i, 0)),
            pl.BlockSpec((EMB_DIM, N_EMB), lambda i: (0, 0)),
        ],
        out_specs=(
            pl.BlockSpec((BLK, N_EMB), lambda i: (i, 0)),
            pl.BlockSpec((1, 1, BLK), lambda i: (i, 0, 0)),
            pl.BlockSpec((1, N_EMB), lambda i: (0, 0)),
            pl.BlockSpec((1, 1), lambda i: (0, 0)),
            pl.BlockSpec((1, 1), lambda i: (0, 0)),
        ),
        out_shape=out_shapes,
        scratch_shapes=[pltpu.SMEM((1, 1), jnp.float32)],
    )(z_flat, emb_t)

    encoding_indices = idx3.reshape(N_ROWS)
    emb_pad = jnp.pad(embedding, ((0, 0), (0, 128 - EMB_DIM)))
    zq = _sc_gather(encoding_indices, emb_pad)[:, :EMB_DIM]
    z_q_out = jnp.transpose(zq.reshape(B, H, W, D), (0, 3, 1, 2))
    return (z_q_out, loss.reshape(()), perp.reshape(()),
            encoding_indices, dist)
